# baseline (speedup 1.0000x reference)
import functools

import jax
import jax.numpy as jnp
from jax import lax
from jax.experimental import pallas as pl
from jax.experimental.pallas import tpu as pltpu
from jax.experimental.pallas import tpu_sc as plsc

_VOCAB_TILE = 3072
_L = 16
_CHUNK = 128


def _sc_info():
    try:
        info = plsc.get_sparse_core_info()
        return info.num_cores, info.num_subcores
    except Exception:
        return 2, 16


@functools.cache
def _make_gather_t(n, T, D, V):
    NC, NS = _sc_info()
    NW = NC * NS
    B = n // T
    slab_c = (D * T) // NW
    per_w = B * slab_c
    assert per_w % _CHUNK == 0 and B % _L == 0 and NW % T == 0
    w_per_t = NW // T
    mesh = plsc.VectorSubcoreMesh(core_axis_name="c", subcore_axis_name="s")

    @functools.partial(
        pl.kernel,
        mesh=mesh,
        out_type=jax.ShapeDtypeStruct((n * D,), jnp.float32),
        scratch_types=[
            pltpu.VMEM((B,), jnp.int32),
            pltpu.VMEM((per_w,), jnp.int32),
            pltpu.VMEM((per_w,), jnp.float32),
            pltpu.SemaphoreType.DMA,
        ],
        compiler_params=pltpu.CompilerParams(use_tc_tiling_on_sc=False),
    )
    def gather_t(idx_hbm, tok1_hbm, x_hbm, idx_v, ivec, x_s, sem):
        wid = lax.axis_index("s") * NC + lax.axis_index("c")
        t = wid // w_per_t
        c0 = (wid % w_per_t) * slab_c
        pltpu.sync_copy(idx_hbm.at[pl.ds(t * B, B)], idx_v)
        for cc in range(slab_c):
            cbase = (c0 + cc) * V
            for a in range(B // _L):
                i16 = idx_v[pl.ds(a * _L, _L)]
                ivec[pl.ds(cc * B + a * _L, _L)] = i16 + cbase
        copies = [
            pltpu.async_copy(
                tok1_hbm.at[ivec.at[pl.ds(k * _CHUNK, _CHUNK)]],
                x_s.at[pl.ds(k * _CHUNK, _CHUNK)],
                sem,
            )
            for k in range(per_w // _CHUNK)
        ]
        for cp in copies:
            cp.wait()
        pltpu.sync_copy(x_s, x_hbm.at[pl.ds(wid * per_w, per_w)])

    return gather_t


def _mm_body(w_ref, x_ref, pos_ref, b_ref, o_ref):
    wt = lax.transpose(w_ref[...], (1, 0))
    vt = o_ref.shape[1]
    row0 = pl.multiple_of(pl.program_id(0) * vt, 128)
    bias = b_ref[pl.ds(row0, vt)].reshape(vt, 1)
    for t in range(o_ref.shape[0]):
        acc = lax.dot_general(
            wt, x_ref[t],
            dimension_numbers=(((1,), (0,)), ((), ())),
            preferred_element_type=jnp.float32,
        )
        pw = lax.dot_general(
            wt, pos_ref[pl.ds(t, 1), :],
            dimension_numbers=(((1,), (1,)), ((), ())),
            preferred_element_type=jnp.float32,
        )
        o_ref[t] = acc + pw + bias


def _matmul_t(W, xT, pos, b1):
    d, V = W.shape
    T, _, B = xT.shape
    vt = _VOCAB_TILE
    return pl.pallas_call(
        _mm_body,
        grid=(pl.cdiv(V, vt),),
        in_specs=[
            pl.BlockSpec((d, vt), lambda j: (0, j)),
            pl.BlockSpec((T, d, B), lambda j: (0, 0, 0)),
            pl.BlockSpec((T, d), lambda j: (0, 0)),
            pl.BlockSpec((V,), lambda j: (0,)),
        ],
        out_specs=pl.BlockSpec((T, vt, B), lambda j: (0, j, 0)),
        out_shape=jax.ShapeDtypeStruct((T, V, B), jnp.float32),
        compiler_params=pltpu.CompilerParams(
            dimension_semantics=("parallel",),
            fuse_transposed_lhs_in_matmul=True,
        ),
    )(W, xT, pos, b1)


def kernel(idx, tok_table, pos_table, W, b):
    B, T = idx.shape
    V, D = tok_table.shape
    n = B * T
    idx_t = idx.T.reshape(n)
    tok1 = tok_table.T.reshape(V * D)
    x_flat = _make_gather_t(n, T, D, V)(idx_t, tok1)
    xT = x_flat.reshape(T, D, B)
    out = _matmul_t(W, xT, pos_table[:T].astype(jnp.float32), b)
    return jnp.transpose(out, (2, 0, 1))
